# Initial kernel scaffold; baseline (speedup 1.0000x reference)
import jax
import jax.numpy as jnp
from jax.experimental import pallas as pl


def kernel(x, edge_index, edge_attr, batch, nonring, nrbidx, action, actor_params, critic_params):
    raise NotImplementedError("write your pallas kernel here")


# speedup vs baseline: 3.8116x; 3.8116x over previous
import functools

import jax
import jax.numpy as jnp
from jax import lax
from jax.experimental import pallas as pl
from jax.experimental.pallas import tpu as pltpu
from jax.experimental.pallas import tpu_sc as plsc

N_NODES = 50000
N_EDGES = 800000
B = 500
NODES_PER = 100
T_PER = 20
T_TOTAL = B * T_PER
DIM = 64
HALF = 32
ACTION_DIM = 6
MP_STEPS = 3
S2S_STEPS = 6

NC = 2
NS = 16
MACRO = 512
KSUB = 4
EPT = 50176
NM = EPT // MACRO
E_PAD = EPT * NS
DUMP = N_NODES
ACC_R = 51200
RPT = ACC_R // NS
ZCH = 640
QUART = 16


def _sc_mesh():
    return plsc.VectorSubcoreMesh(core_axis_name="c", subcore_axis_name="s",
                                  num_cores=NC, num_subcores=NS)


@functools.lru_cache(maxsize=None)
def _make_edge_kernel(p):
    def _edge_body(idx_hbm, hw_hbm, ea_hbm, out_hbm,
                   idx_v, dst_v, g_v, ea_v, zbuf, acc, sem_in, sem_g):
        c = lax.axis_index("c")
        s = lax.axis_index("s")
        q = 2 * p + c

        def zb(i, carry):
            zbuf[i, :] = jnp.zeros((16,), jnp.float32)
            return carry

        lax.fori_loop(0, ZCH, zb, 0)
        for k in range(RPT // ZCH):
            pltpu.sync_copy(zbuf, acc.at[pl.ds(s * RPT + k * ZCH, ZCH)])
        plsc.subcore_barrier()

        def macro_body(m, carry):
            e0 = s * EPT + m * MACRO
            r0 = s * (EPT // 128) + m * KSUB
            cp_i = pltpu.async_copy(idx_hbm.at[q, pl.ds(r0, KSUB)], idx_v, sem_in)
            cp_d = pltpu.async_copy(idx_hbm.at[4, pl.ds(r0, KSUB)], dst_v, sem_in)
            cp_e = pltpu.async_copy(ea_hbm.at[q, pl.ds(e0, MACRO)], ea_v, sem_g)
            cp_i.wait()
            gs = [pltpu.async_copy(hw_hbm.at[idx_v.at[k]],
                                   g_v.at[pl.ds(k * 128, 128)], sem_g)
                  for k in range(KSUB)]
            cp_d.wait()
            for g in gs:
                g.wait()
            cp_e.wait()

            def comp(i, carry2):
                base = i * 8
                for r in range(8):
                    g_v[base + r, :] = jnp.maximum(
                        g_v[base + r, :] + ea_v[base + r, :], 0.0)
                return carry2

            lax.fori_loop(0, MACRO // 8, comp, 0)
            for k in range(KSUB):
                pltpu.sync_copy(g_v.at[pl.ds(k * 128, 128)],
                                acc.at[dst_v.at[k]], add=True)
            return carry

        lax.fori_loop(0, NM, macro_body, 0)
        plsc.subcore_barrier()
        for k in range(RPT // ZCH):
            r0 = s * RPT + k * ZCH
            pltpu.sync_copy(acc.at[pl.ds(r0, ZCH)], zbuf)
            pltpu.sync_copy(zbuf, out_hbm.at[c, pl.ds(r0, ZCH)])

    return functools.partial(
        pl.kernel,
        out_type=jax.ShapeDtypeStruct((NC, ACC_R, QUART), jnp.float32),
        mesh=_sc_mesh(),
        compiler_params=pltpu.CompilerParams(use_tc_tiling_on_sc=False),
        scratch_types=[
            pltpu.VMEM((KSUB, 128), jnp.int32),
            pltpu.VMEM((KSUB, 128), jnp.int32),
            pltpu.VMEM((MACRO, QUART), jnp.float32),
            pltpu.VMEM((MACRO, QUART), jnp.float32),
            pltpu.VMEM((ZCH, QUART), jnp.float32),
            pltpu.VMEM_SHARED((ACC_R, QUART), jnp.float32),
            pltpu.SemaphoreType.DMA,
            pltpu.SemaphoreType.DMA,
        ],
    )(_edge_body)


def _edge_kernel(idx3, hw4, ea4, p):
    return _make_edge_kernel(p)(idx3, hw4, ea4)


NF_PAD = 40960
NF_PER_TILE = NF_PAD // (NC * NS)


@functools.lru_cache(maxsize=None)
def _make_nf_gather():
    return functools.partial(
        pl.kernel,
        out_type=jax.ShapeDtypeStruct((NF_PAD, DIM), jnp.float32),
        mesh=_sc_mesh(),
        compiler_params=pltpu.CompilerParams(use_tc_tiling_on_sc=False),
        scratch_types=[
            pltpu.VMEM((128,), jnp.int32),
            pltpu.VMEM((128, DIM), jnp.float32),
            pltpu.SemaphoreType.DMA,
        ],
    )(_nf_body)


def _nf_gather(idx, h):
    return _make_nf_gather()(idx, h)


def _nf_body(idx_hbm, h_hbm, out_hbm, idx_v, row_v, sem):
    c = lax.axis_index("c")
    s = lax.axis_index("s")
    w = s * NC + c

    def body(i, carry):
        base = w * NF_PER_TILE + i * 128
        pltpu.sync_copy(idx_hbm.at[pl.ds(base, 128)], idx_v)
        pltpu.async_copy(h_hbm.at[idx_v], row_v, sem).wait()
        pltpu.sync_copy(row_v, out_hbm.at[pl.ds(base, 128)])
        return carry

    lax.fori_loop(0, NF_PER_TILE // 128, body, 0)


NB = 2000
N_NBLK = N_NODES // NB
EB = 4096
N_EBLK = E_PAD // EB


def _embed_body(x_ref, ew_ref, eb_ref, wh_ref, h_ref, hw_ref):
    h = jnp.maximum(jnp.dot(x_ref[...], ew_ref[...],
                            preferred_element_type=jnp.float32) + eb_ref[...], 0.0)
    h_ref[...] = h
    hw_ref[...] = jnp.dot(h, wh_ref[...], preferred_element_type=jnp.float32)


def _embed(x, ew, ebias, wh):
    return pl.pallas_call(
        _embed_body,
        grid=(N_NBLK,),
        in_specs=[
            pl.BlockSpec((NB, 3), lambda i: (i, 0)),
            pl.BlockSpec((3, DIM), lambda i: (0, 0)),
            pl.BlockSpec((1, DIM), lambda i: (0, 0)),
            pl.BlockSpec((DIM, DIM), lambda i: (0, 0)),
        ],
        out_specs=[
            pl.BlockSpec((NB, DIM), lambda i: (i, 0)),
            pl.BlockSpec((NB, DIM), lambda i: (i, 0)),
        ],
        out_shape=[
            jax.ShapeDtypeStruct((N_NODES, DIM), jnp.float32),
            jax.ShapeDtypeStruct((N_NODES, DIM), jnp.float32),
        ],
    )(x, ew, ebias, wh)


def _ea_body(ea_ref, w_ref, b_ref, out_ref):
    r = jnp.dot(ea_ref[...], w_ref[...],
                preferred_element_type=jnp.float32) + b_ref[...]
    for q in range(4):
        out_ref[q] = r[:, q * QUART:(q + 1) * QUART]


def _ea_proj(ea8, w8, bias):
    return pl.pallas_call(
        _ea_body,
        grid=(N_EBLK,),
        in_specs=[
            pl.BlockSpec((EB, 8), lambda i: (i, 0)),
            pl.BlockSpec((8, DIM), lambda i: (0, 0)),
            pl.BlockSpec((1, DIM), lambda i: (0, 0)),
        ],
        out_specs=pl.BlockSpec((4, EB, QUART), lambda i: (0, i, 0)),
        out_shape=jax.ShapeDtypeStruct((4, E_PAD, QUART), jnp.float32),
    )(ea8, w8, bias)


def _gru_body(a_ref, b_ref2, h_ref, wih_ref, whh_ref, bih_ref, bhh_ref, wh_ref,
              hn_ref, hwn_ref):
    h = h_ref[...]
    xg = (jnp.dot(a_ref[0], wih_ref[:QUART], preferred_element_type=jnp.float32)
          + jnp.dot(a_ref[1], wih_ref[QUART:2 * QUART],
                    preferred_element_type=jnp.float32)
          + jnp.dot(b_ref2[0], wih_ref[2 * QUART:3 * QUART],
                    preferred_element_type=jnp.float32)
          + jnp.dot(b_ref2[1], wih_ref[3 * QUART:],
                    preferred_element_type=jnp.float32)
          + bih_ref[...])
    hg = jnp.dot(h, whh_ref[...], preferred_element_type=jnp.float32) + bhh_ref[...]
    r = jax.nn.sigmoid(xg[:, :DIM] + hg[:, :DIM])
    z = jax.nn.sigmoid(xg[:, DIM:2 * DIM] + hg[:, DIM:2 * DIM])
    n = jnp.tanh(xg[:, 2 * DIM:] + r * hg[:, 2 * DIM:])
    hn = (1.0 - z) * n + z * h
    hn_ref[...] = hn
    hwn_ref[...] = jnp.dot(hn, wh_ref[...], preferred_element_type=jnp.float32)


def _gru(agg_a, agg_b, h, wih, whh, bih, bhh, wh):
    return pl.pallas_call(
        _gru_body,
        grid=(N_NBLK,),
        in_specs=[
            pl.BlockSpec((NC, NB, QUART), lambda i: (0, i, 0)),
            pl.BlockSpec((NC, NB, QUART), lambda i: (0, i, 0)),
            pl.BlockSpec((NB, DIM), lambda i: (i, 0)),
            pl.BlockSpec((DIM, 3 * DIM), lambda i: (0, 0)),
            pl.BlockSpec((DIM, 3 * DIM), lambda i: (0, 0)),
            pl.BlockSpec((1, 3 * DIM), lambda i: (0, 0)),
            pl.BlockSpec((1, 3 * DIM), lambda i: (0, 0)),
            pl.BlockSpec((DIM, DIM), lambda i: (0, 0)),
        ],
        out_specs=[
            pl.BlockSpec((NB, DIM), lambda i: (i, 0)),
            pl.BlockSpec((NB, DIM), lambda i: (i, 0)),
        ],
        out_shape=[
            jax.ShapeDtypeStruct((N_NODES, DIM), jnp.float32),
            jax.ShapeDtypeStruct((N_NODES, DIM), jnp.float32),
        ],
    )(agg_a, agg_b, h, wih, whh, bih, bhh, wh)


GB = 100
N_GBLK = B // GB


def _s2s_body(xn_ref, wih_ref, whh_ref, b_ref, out_ref):
    xn = xn_ref[...]
    h = jnp.zeros((GB, DIM), jnp.float32)
    cc = jnp.zeros((GB, DIM), jnp.float32)
    q_star = jnp.zeros((GB, 2 * DIM), jnp.float32)
    for _ in range(S2S_STEPS):
        gates = (jnp.dot(q_star, wih_ref[...], preferred_element_type=jnp.float32)
                 + jnp.dot(h, whh_ref[...], preferred_element_type=jnp.float32)
                 + b_ref[...])
        ig = jax.nn.sigmoid(gates[:, :DIM])
        fg = jax.nn.sigmoid(gates[:, DIM:2 * DIM])
        gg = jnp.tanh(gates[:, 2 * DIM:3 * DIM])
        og = jax.nn.sigmoid(gates[:, 3 * DIM:])
        cc = fg * cc + ig * gg
        h = og * jnp.tanh(cc)
        e = jnp.sum(xn * h[:, None, :], axis=-1)
        emax = jnp.max(e, axis=1, keepdims=True)
        ee = jnp.exp(e - emax)
        a = ee / jnp.sum(ee, axis=1, keepdims=True)
        r = jnp.sum(a[:, :, None] * xn, axis=1)
        q_star = jnp.concatenate([h, r], axis=-1)
    out_ref[0] = q_star


def _set2set(xn3, wih, whh, bias):
    out = pl.pallas_call(
        _s2s_body,
        grid=(N_GBLK,),
        in_specs=[
            pl.BlockSpec((GB, NODES_PER, DIM), lambda i: (i, 0, 0)),
            pl.BlockSpec((2 * DIM, 4 * DIM), lambda i: (0, 0)),
            pl.BlockSpec((DIM, 4 * DIM), lambda i: (0, 0)),
            pl.BlockSpec((1, 4 * DIM), lambda i: (0, 0)),
        ],
        out_specs=pl.BlockSpec((1, GB, 2 * DIM), lambda i: (i, 0, 0)),
        out_shape=jax.ShapeDtypeStruct((N_GBLK, GB, 2 * DIM), jnp.float32),
    )(xn3, wih, whh, bias)
    return out.reshape(B, 2 * DIM)


TB = GB * T_PER


def _actor_body(qs_ref, fcw_ref, fcb_ref, nf_ref, m1a_ref, m1b_ref, m1b2_ref,
                m2_ref, m2b_ref, act_ref, lp_ref, ent_ref):
    ge = jnp.dot(qs_ref[0], fcw_ref[...],
                 preferred_element_type=jnp.float32) + fcb_ref[...]
    ge_rep = jnp.broadcast_to(ge[:, None, :], (GB, T_PER, DIM)).reshape(TB, DIM)
    hid = jnp.maximum(
        jnp.dot(ge_rep, m1a_ref[...], preferred_element_type=jnp.float32)
        + jnp.dot(nf_ref[...], m1b_ref[...], preferred_element_type=jnp.float32)
        + m1b2_ref[...], 0.0)
    logits = jnp.dot(hid, m2_ref[...],
                     preferred_element_type=jnp.float32) + m2b_ref[...]
    m = jnp.max(logits, axis=-1, keepdims=True)
    sh = logits - m
    lse = jnp.log(jnp.sum(jnp.exp(sh), axis=-1, keepdims=True))
    lp = sh - lse
    lp3 = lp.reshape(GB, T_PER, ACTION_DIM)
    act = act_ref[0]
    lpa = jnp.zeros((GB, T_PER), jnp.float32)
    ent = jnp.zeros((GB, T_PER), jnp.float32)
    for k in range(ACTION_DIM):
        lk = lp3[:, :, k]
        lpa = lpa + jnp.where(act == k, lk, 0.0)
        ent = ent - jnp.exp(lk) * lk
    lp_ref[0] = lpa
    ent_ref[0] = ent


def _actor_head(q_star, fcw, fcb, nf, m1a, m1b, m1bias, m2, m2b, action):
    qs3 = q_star.reshape(N_GBLK, GB, 2 * DIM)
    act3 = action.reshape(N_GBLK, GB, T_PER)
    lp, ent = pl.pallas_call(
        _actor_body,
        grid=(N_GBLK,),
        in_specs=[
            pl.BlockSpec((1, GB, 2 * DIM), lambda i: (i, 0, 0)),
            pl.BlockSpec((2 * DIM, DIM), lambda i: (0, 0)),
            pl.BlockSpec((1, DIM), lambda i: (0, 0)),
            pl.BlockSpec((TB, 4 * DIM), lambda i: (i, 0)),
            pl.BlockSpec((DIM, DIM), lambda i: (0, 0)),
            pl.BlockSpec((4 * DIM, DIM), lambda i: (0, 0)),
            pl.BlockSpec((1, DIM), lambda i: (0, 0)),
            pl.BlockSpec((DIM, ACTION_DIM), lambda i: (0, 0)),
            pl.BlockSpec((1, ACTION_DIM), lambda i: (0, 0)),
            pl.BlockSpec((1, GB, T_PER), lambda i: (i, 0, 0)),
        ],
        out_specs=[
            pl.BlockSpec((1, GB, T_PER), lambda i: (i, 0, 0)),
            pl.BlockSpec((1, GB, T_PER), lambda i: (i, 0, 0)),
        ],
        out_shape=[
            jax.ShapeDtypeStruct((N_GBLK, GB, T_PER), jnp.float32),
            jax.ShapeDtypeStruct((N_GBLK, GB, T_PER), jnp.float32),
        ],
    )(qs3, fcw, fcb, nf, m1a, m1b, m1bias, m2, m2b, act3)
    return lp.reshape(B, T_PER), ent.reshape(B, T_PER)


def _critic_body(qs_ref, w1_ref, b1_ref, w2_ref, b2_ref, w3_ref, b3_ref, v_ref):
    hc = jnp.maximum(jnp.dot(qs_ref[...], w1_ref[...],
                             preferred_element_type=jnp.float32) + b1_ref[...], 0.0)
    hc = jnp.maximum(jnp.dot(hc, w2_ref[...],
                             preferred_element_type=jnp.float32) + b2_ref[...], 0.0)
    v_ref[...] = jnp.dot(hc, w3_ref[...],
                         preferred_element_type=jnp.float32) + b3_ref[...]


def _critic_head(q_star, w1, b1, w2, b2, w3, b3):
    return pl.pallas_call(
        _critic_body,
        out_shape=jax.ShapeDtypeStruct((B, 1), jnp.float32),
    )(q_star, w1, b1, w2, b2, w3, b3)


def _row2(v):
    return v.reshape(1, -1)


def _run_net(x, idx3, ea8, p):
    wh = p['edge_W'][:DIM]
    we8 = jnp.concatenate([p['edge_W'][DIM:], jnp.zeros((1, DIM), jnp.float32)], 0)
    h, hw = _embed(x, p['emb_W'], _row2(p['emb_b']), wh)
    ea2 = _ea_proj(ea8, we8, _row2(p['edge_b']))
    for _ in range(MP_STEPS):
        hw4 = hw.reshape(4 * N_NODES, QUART)
        agg_a = _edge_kernel(idx3, hw4, ea2, 0)
        agg_b = _edge_kernel(idx3, hw4, ea2, 1)
        h, hw = _gru(agg_a, agg_b, h, p['gru_Wih'], p['gru_Whh'],
                     _row2(p['gru_bih']), _row2(p['gru_bhh']), wh)
    q_star = _set2set(h.reshape(B, NODES_PER, DIM), p['lstm_Wih'],
                      p['lstm_Whh'], _row2(p['lstm_b']))
    return h, q_star


def kernel(x, edge_index, edge_attr, batch, nonring, nrbidx, action,
           actor_params, critic_params):
    src = edge_index[0]
    dst = edge_index[1]
    pad = E_PAD - N_EDGES
    dstp = jnp.pad(dst, (0, pad), constant_values=DUMP)
    idx = jnp.stack([4 * src, 4 * src + 1, 4 * src + 2, 4 * src + 3])
    idx3 = jnp.concatenate(
        [jnp.pad(idx, ((0, 0), (0, pad))), dstp[None]], 0
    ).reshape(5, E_PAD // 128, 128)
    ea8 = jnp.pad(edge_attr, ((0, pad), (0, 1)))

    h_a, qs_a = _run_net(x, idx3, ea8, actor_params)
    h_c, qs_c = _run_net(x, idx3, ea8, critic_params)

    nr_flat = jnp.pad(nonring.reshape(-1), (0, NF_PAD - 4 * T_TOTAL))
    nf = _nf_gather(nr_flat, h_a).reshape(NF_PAD // 4, 4 * DIM)

    ap = actor_params
    log_pi, ent = _actor_head(
        qs_a, ap['fc_W'], _row2(ap['fc_b']), nf,
        ap['mlp1_W'][:DIM], ap['mlp1_W'][DIM:], _row2(ap['mlp1_b']),
        ap['mlp2_W'], _row2(ap['mlp2_b']), action)

    cp = critic_params
    v = _critic_head(qs_c, cp['c1_W'], _row2(cp['c1_b']),
                     cp['c2_W'], _row2(cp['c2_b']),
                     cp['c3_W'], _row2(cp['c3_b']))
    return (log_pi, ent, v)
